# baseline (speedup 1.0000x reference)
import functools

import jax
import jax.numpy as jnp
from jax import lax
from jax.experimental import pallas as pl
from jax.experimental.pallas import tpu as pltpu
from jax.experimental.pallas import tpu_sc as plsc

F32 = jnp.float32
I32 = jnp.int32

ALPHA = 0.1
K_ITERS = 10

NC = 2
NS = 16
LANES = 16

TILE_N = 6272
N_PAD = NS * TILE_N

BATCH = 6144
LOAD_E = 6144
JB = LOAD_E // BATCH


def _mlp_body(x_ref, w1_ref, b1_ref, w2_ref, b2_ref, o_ref):
    h = jnp.dot(x_ref[...], w1_ref[...], preferred_element_type=F32)
    h = jnp.maximum(h + b1_ref[...], 0.0)
    o_ref[...] = jnp.dot(h, w2_ref[...], preferred_element_type=F32) + b2_ref[...]


def _mlp(x, W1, b1, W2, b2):
    N, IN = x.shape
    HID = W1.shape[1]
    OUT = W2.shape[1]
    BM = 2000
    assert N % BM == 0
    return pl.pallas_call(
        _mlp_body,
        grid=(N // BM,),
        in_specs=[
            pl.BlockSpec((BM, IN), lambda i: (i, 0)),
            pl.BlockSpec((IN, HID), lambda i: (0, 0)),
            pl.BlockSpec((1, HID), lambda i: (0, 0)),
            pl.BlockSpec((HID, OUT), lambda i: (0, 0)),
            pl.BlockSpec((1, OUT), lambda i: (0, 0)),
        ],
        out_specs=pl.BlockSpec((BM, OUT), lambda i: (i, 0)),
        out_shape=jax.ShapeDtypeStruct((N, OUT), F32),
    )(x, W1, b1.reshape(1, HID), W2, b2.reshape(1, OUT))


def _make_appnp(E_PAD, NCHUNK):
    rows_per_load = LOAD_E // BATCH
    NROWS = E_PAD // BATCH
    tile_rows = NCHUNK * rows_per_load
    assert tile_rows * NS == NROWS

    mesh = plsc.VectorSubcoreMesh(
        core_axis_name="c", subcore_axis_name="s", num_cores=NC, num_subcores=NS
    )

    @functools.partial(
        pl.kernel,
        out_type=jax.ShapeDtypeStruct((NC, N_PAD), F32),
        mesh=mesh,
        scratch_types=[
            pltpu.VMEM_SHARED((N_PAD,), F32),
            pltpu.VMEM_SHARED((N_PAD,), F32),
            pltpu.VMEM_SHARED((N_PAD,), F32),
            pltpu.VMEM_SHARED((N_PAD,), F32),
            pltpu.VMEM((JB, BATCH), I32),
            pltpu.VMEM((JB, BATCH), I32),
            pltpu.VMEM((JB, BATCH), I32),
            pltpu.VMEM((JB, BATCH), I32),
            pltpu.VMEM((JB, BATCH), I32),
            pltpu.VMEM((JB, BATCH), I32),
            pltpu.VMEM((JB, BATCH), I32),
            pltpu.VMEM((JB, BATCH), I32),
            pltpu.VMEM((JB, BATCH), F32),
            pltpu.VMEM((JB, BATCH), F32),
            pltpu.VMEM((JB, BATCH), F32),
            pltpu.VMEM((JB, BATCH), F32),
            pltpu.VMEM((TILE_N,), F32),
            pltpu.VMEM((TILE_N,), F32),
            pltpu.VMEM((TILE_N,), F32),
            pltpu.VMEM((TILE_N,), F32),
            pltpu.SemaphoreType.DMA((4,)),
            pltpu.SemaphoreType.DMA((4,)),
            pltpu.SemaphoreType.DMA((4,)),
        ],
    )
    def appnp(zt, col2, row2, val2, out,
              Hs, Ss, Zs, Ds, col_v0, col_v1, col_v2, col_v3,
              row_v0, row_v1, row_v2, row_v3,
              g_v0, g_v1, g_v2, g_v3, u_a, u_b, u_c, zz, sem_l, sem_g, sem_s):
        v_v = g_v0
        col_b = [col_v0, col_v1, col_v2, col_v3]
        row_b = [row_v0, row_v1, row_v2, row_v3]
        g_b = [g_v0, g_v1, g_v2, g_v3]
        row_v = row_v0
        c = lax.axis_index("c")
        s = lax.axis_index("s")
        node0 = s * TILE_N
        row0 = s * tile_rows

        def fill_zero(t, _):
            zz[pl.ds(t * LANES, LANES)] = jnp.zeros((LANES,), F32)
            return 0
        lax.fori_loop(0, TILE_N // LANES, fill_zero, 0)
        pltpu.sync_copy(zz, Ss.at[pl.ds(node0, TILE_N)])
        pltpu.sync_copy(zz, Ds.at[pl.ds(node0, TILE_N)])

        pltpu.sync_copy(zt.at[c].at[pl.ds(node0, TILE_N)], u_a)
        pltpu.sync_copy(u_a, Zs.at[pl.ds(node0, TILE_N)])
        pltpu.sync_copy(u_a, Hs.at[pl.ds(node0, TILE_N)])

        plsc.subcore_barrier()

        def dchunk(i, _):
            r0 = row0 + i * rows_per_load
            pltpu.sync_copy(row2.at[pl.ds(r0, rows_per_load)], row_v)
            pltpu.sync_copy(val2.at[pl.ds(r0, rows_per_load)], v_v)
            def dj(j, _):
                pltpu.sync_copy(v_v.at[j], Ds.at[row_v.at[j]])
                return 0
            lax.fori_loop(0, JB, dj, 0)
            return 0
        lax.fori_loop(0, NCHUNK, dchunk, 0)

        plsc.subcore_barrier()

        def one_iter(k, _):
            gat = [None, None, None, None]
            sca = [None, None, None, None]
            for i in range(NCHUNK):
                b = i % 4
                if sca[b] is not None:
                    sca[b].wait()
                r0 = row0 + i * rows_per_load
                lc = pltpu.async_copy(col2.at[pl.ds(r0, rows_per_load)],
                                      col_b[b], sem_l.at[b])
                lr = pltpu.async_copy(row2.at[pl.ds(r0, rows_per_load)],
                                      row_b[b], sem_l.at[b])
                if i > 0:
                    pb = (i - 1) % 4
                    gat[pb].wait()
                    sca[pb] = pltpu.async_copy(g_b[pb].at[0],
                                               Ss.at[row_b[pb].at[0]],
                                               sem_s.at[pb], add=True)
                lc.wait()
                lr.wait()
                gat[b] = pltpu.async_copy(Hs.at[col_b[b].at[0]], g_b[b].at[0],
                                          sem_g.at[b])
            lastb = (NCHUNK - 1) % 4
            gat[lastb].wait()
            sca_last = pltpu.async_copy(g_b[lastb].at[0],
                                        Ss.at[row_b[lastb].at[0]],
                                        sem_s.at[lastb], add=True)
            for cidx in range(max(0, NCHUNK - 4), NCHUNK - 1):
                sca[cidx % 4].wait()
            sca_last.wait()

            plsc.subcore_barrier()

            pltpu.sync_copy(Zs.at[pl.ds(node0, TILE_N)], u_a)
            pltpu.sync_copy(Ss.at[pl.ds(node0, TILE_N)], u_b)
            pltpu.sync_copy(Ds.at[pl.ds(node0, TILE_N)], u_c)
            def upd(t, _):
                sl = pl.ds(t * LANES, LANES)
                u_b[sl] = ALPHA * u_a[sl] + (1.0 - ALPHA) * (u_c[sl] * u_b[sl])
                return 0
            lax.fori_loop(0, TILE_N // LANES, upd, 0)
            pltpu.sync_copy(u_b, Hs.at[pl.ds(node0, TILE_N)])
            pltpu.sync_copy(zz, Ss.at[pl.ds(node0, TILE_N)])

            plsc.subcore_barrier()
            return 0

        lax.fori_loop(0, K_ITERS, one_iter, 0)

        pltpu.sync_copy(Hs.at[pl.ds(node0, TILE_N)], u_a)
        pltpu.sync_copy(u_a, out.at[c].at[pl.ds(node0, TILE_N)])

    return appnp


def kernel(x, W1, b1, W2, b2, edge_index, edge_vals):
    N = x.shape[0]
    E = edge_index.shape[1]

    Z = _mlp(x, W1, b1, W2, b2)
    zt = jnp.zeros((NC, N_PAD), F32).at[:, :N].set(Z.T)

    per_tile = -(-E // (NS * LOAD_E)) * LOAD_E
    E_PAD = per_tile * NS
    NCHUNK = per_tile // LOAD_E
    pad = E_PAD - E

    row = edge_index[0]
    col = edge_index[1]
    if pad:
        pad_idx = jnp.arange(pad, dtype=I32)
        colp = jnp.concatenate([col, pad_idx % N])
        rowp = jnp.concatenate([row, N + pad_idx % (N_PAD - N)])
        valp = jnp.concatenate([edge_vals, jnp.zeros((pad,), F32)])
    else:
        colp, rowp, valp = col, row, edge_vals

    col2 = colp.reshape(-1, BATCH)
    row2 = rowp.reshape(-1, BATCH)
    val2 = valp.reshape(-1, BATCH)

    appnp = _make_appnp(E_PAD, NCHUNK)
    ht = appnp(zt, col2, row2, val2)
    return ht[:, :N].T


# speedup vs baseline: 1.0038x; 1.0038x over previous
import functools

import jax
import jax.numpy as jnp
from jax import lax
from jax.experimental import pallas as pl
from jax.experimental.pallas import tpu as pltpu
from jax.experimental.pallas import tpu_sc as plsc

F32 = jnp.float32
I32 = jnp.int32

ALPHA = 0.1
K_ITERS = 10

NC = 2
NS = 16
LANES = 16

TILE_N = 6272
N_PAD = NS * TILE_N

BATCH = 8192
LOAD_E = 8192
JB = LOAD_E // BATCH


def _mlp_body(x_ref, w1_ref, b1_ref, w2_ref, b2_ref, o_ref):
    h = jnp.dot(x_ref[...], w1_ref[...], preferred_element_type=F32)
    h = jnp.maximum(h + b1_ref[...], 0.0)
    o_ref[...] = jnp.dot(h, w2_ref[...], preferred_element_type=F32) + b2_ref[...]


def _mlp(x, W1, b1, W2, b2):
    N, IN = x.shape
    HID = W1.shape[1]
    OUT = W2.shape[1]
    BM = 2000
    assert N % BM == 0
    return pl.pallas_call(
        _mlp_body,
        grid=(N // BM,),
        in_specs=[
            pl.BlockSpec((BM, IN), lambda i: (i, 0)),
            pl.BlockSpec((IN, HID), lambda i: (0, 0)),
            pl.BlockSpec((1, HID), lambda i: (0, 0)),
            pl.BlockSpec((HID, OUT), lambda i: (0, 0)),
            pl.BlockSpec((1, OUT), lambda i: (0, 0)),
        ],
        out_specs=pl.BlockSpec((BM, OUT), lambda i: (i, 0)),
        out_shape=jax.ShapeDtypeStruct((N, OUT), F32),
    )(x, W1, b1.reshape(1, HID), W2, b2.reshape(1, OUT))


def _make_appnp(E_PAD, NCHUNK):
    rows_per_load = LOAD_E // BATCH
    NROWS = E_PAD // BATCH
    tile_rows = NCHUNK * rows_per_load
    assert tile_rows * NS == NROWS

    mesh = plsc.VectorSubcoreMesh(
        core_axis_name="c", subcore_axis_name="s", num_cores=NC, num_subcores=NS
    )

    @functools.partial(
        pl.kernel,
        out_type=jax.ShapeDtypeStruct((NC, N_PAD), F32),
        mesh=mesh,
        scratch_types=[
            pltpu.VMEM_SHARED((N_PAD,), F32),
            pltpu.VMEM_SHARED((N_PAD,), F32),
            pltpu.VMEM_SHARED((N_PAD,), F32),
            pltpu.VMEM_SHARED((N_PAD,), F32),
            pltpu.VMEM((JB, BATCH), I32),
            pltpu.VMEM((JB, BATCH), I32),
            pltpu.VMEM((JB, BATCH), I32),
            pltpu.VMEM((JB, BATCH), I32),
            pltpu.VMEM((JB, BATCH), I32),
            pltpu.VMEM((JB, BATCH), I32),
            pltpu.VMEM((JB, BATCH), F32),
            pltpu.VMEM((JB, BATCH), F32),
            pltpu.VMEM((JB, BATCH), F32),
            pltpu.VMEM((TILE_N,), F32),
            pltpu.VMEM((TILE_N,), F32),
            pltpu.VMEM((TILE_N,), F32),
            pltpu.VMEM((TILE_N,), F32),
            pltpu.SemaphoreType.DMA((3,)),
            pltpu.SemaphoreType.DMA((3,)),
            pltpu.SemaphoreType.DMA((3,)),
        ],
    )
    def appnp(zt, col2, row2, val2, out,
              Hs, Ss, Zs, Ds, col_v0, col_v1, col_v2, row_v0, row_v1, row_v2,
              g_v0, g_v1, g_v2, u_a, u_b, u_c, zz, sem_l, sem_g, sem_s):
        v_v = g_v0
        col_b = [col_v0, col_v1, col_v2]
        row_b = [row_v0, row_v1, row_v2]
        g_b = [g_v0, g_v1, g_v2]
        row_v = row_v0
        c = lax.axis_index("c")
        s = lax.axis_index("s")
        node0 = s * TILE_N
        row0 = s * tile_rows

        def fill_zero(t, _):
            zz[pl.ds(t * LANES, LANES)] = jnp.zeros((LANES,), F32)
            return 0
        lax.fori_loop(0, TILE_N // LANES, fill_zero, 0)
        pltpu.sync_copy(zz, Ss.at[pl.ds(node0, TILE_N)])
        pltpu.sync_copy(zz, Ds.at[pl.ds(node0, TILE_N)])

        pltpu.sync_copy(zt.at[c].at[pl.ds(node0, TILE_N)], u_a)
        pltpu.sync_copy(u_a, Zs.at[pl.ds(node0, TILE_N)])
        pltpu.sync_copy(u_a, Hs.at[pl.ds(node0, TILE_N)])

        plsc.subcore_barrier()

        def dchunk(i, _):
            r0 = row0 + i * rows_per_load
            pltpu.sync_copy(row2.at[pl.ds(r0, rows_per_load)], row_v)
            pltpu.sync_copy(val2.at[pl.ds(r0, rows_per_load)], v_v)
            def dj(j, _):
                pltpu.sync_copy(v_v.at[j], Ds.at[row_v.at[j]])
                return 0
            lax.fori_loop(0, JB, dj, 0)
            return 0
        lax.fori_loop(0, NCHUNK, dchunk, 0)

        plsc.subcore_barrier()

        def one_iter(k, _):
            gat = [None, None, None]
            sca = [None, None, None]
            for i in range(NCHUNK):
                b = i % 3
                if sca[b] is not None:
                    sca[b].wait()
                r0 = row0 + i * rows_per_load
                lc = pltpu.async_copy(col2.at[pl.ds(r0, rows_per_load)],
                                      col_b[b], sem_l.at[b])
                lr = pltpu.async_copy(row2.at[pl.ds(r0, rows_per_load)],
                                      row_b[b], sem_l.at[b])
                if i > 0:
                    pb = (i - 1) % 3
                    gat[pb].wait()
                    sca[pb] = pltpu.async_copy(g_b[pb].at[0],
                                               Ss.at[row_b[pb].at[0]],
                                               sem_s.at[pb], add=True)
                lc.wait()
                lr.wait()
                gat[b] = pltpu.async_copy(Hs.at[col_b[b].at[0]], g_b[b].at[0],
                                          sem_g.at[b])
            lastb = (NCHUNK - 1) % 3
            gat[lastb].wait()
            sca_last = pltpu.async_copy(g_b[lastb].at[0],
                                        Ss.at[row_b[lastb].at[0]],
                                        sem_s.at[lastb], add=True)
            for cidx in range(max(0, NCHUNK - 3), NCHUNK - 1):
                sca[cidx % 3].wait()
            sca_last.wait()

            plsc.subcore_barrier()

            pltpu.sync_copy(Zs.at[pl.ds(node0, TILE_N)], u_a)
            pltpu.sync_copy(Ss.at[pl.ds(node0, TILE_N)], u_b)
            pltpu.sync_copy(Ds.at[pl.ds(node0, TILE_N)], u_c)
            def upd(t, _):
                sl = pl.ds(t * LANES, LANES)
                u_b[sl] = ALPHA * u_a[sl] + (1.0 - ALPHA) * (u_c[sl] * u_b[sl])
                return 0
            lax.fori_loop(0, TILE_N // LANES, upd, 0)
            pltpu.sync_copy(u_b, Hs.at[pl.ds(node0, TILE_N)])
            pltpu.sync_copy(zz, Ss.at[pl.ds(node0, TILE_N)])

            plsc.subcore_barrier()
            return 0

        lax.fori_loop(0, K_ITERS, one_iter, 0)

        pltpu.sync_copy(Hs.at[pl.ds(node0, TILE_N)], u_a)
        pltpu.sync_copy(u_a, out.at[c].at[pl.ds(node0, TILE_N)])

    return appnp


def kernel(x, W1, b1, W2, b2, edge_index, edge_vals):
    N = x.shape[0]
    E = edge_index.shape[1]

    Z = _mlp(x, W1, b1, W2, b2)
    zt = jnp.zeros((NC, N_PAD), F32).at[:, :N].set(Z.T)

    per_tile = -(-E // (NS * LOAD_E)) * LOAD_E
    E_PAD = per_tile * NS
    NCHUNK = per_tile // LOAD_E
    pad = E_PAD - E

    row = edge_index[0]
    col = edge_index[1]
    if pad:
        pad_idx = jnp.arange(pad, dtype=I32)
        colp = jnp.concatenate([col, pad_idx % N])
        rowp = jnp.concatenate([row, N + pad_idx % (N_PAD - N)])
        valp = jnp.concatenate([edge_vals, jnp.zeros((pad,), F32)])
    else:
        colp, rowp, valp = col, row, edge_vals

    col2 = colp.reshape(-1, BATCH)
    row2 = rowp.reshape(-1, BATCH)
    val2 = valp.reshape(-1, BATCH)

    appnp = _make_appnp(E_PAD, NCHUNK)
    ht = appnp(zt, col2, row2, val2)
    return ht[:, :N].T
